# Initial kernel scaffold; baseline (speedup 1.0000x reference)
import jax
import jax.numpy as jnp
from jax.experimental import pallas as pl


def kernel(in_feat, edge_index, Mw0, Mb0, Uw0, Ub0, MixW0, Mixb0, Mw1, Mb1, Uw1, Ub1, MixW1, Mixb1, Mw2, Mb2, Uw2, Ub2, MixW2, Mixb2, FCw, FCb):
    raise NotImplementedError("write your pallas kernel here")


# speedup vs baseline: 1.1530x; 1.1530x over previous
import functools
import jax
import jax.numpy as jnp
from jax.experimental import pallas as pl
from jax.experimental.pallas import tpu as pltpu

D = 128
DELTA = 3.5
NP = 10240
R = 1024
GRID = NP // R


def _proj_body(h_ref, mw_ref, mb_ref, a_ref, b_ref):
    h = h_ref[...]
    a_ref[...] = jnp.dot(h, mw_ref[0:D, :], preferred_element_type=jnp.float32)
    b_ref[...] = jnp.dot(h, mw_ref[D:2 * D, :], preferred_element_type=jnp.float32) + mb_ref[...]


def _tc_proj(h, Mw, Mb):
    blk = lambda i: (i, 0)
    full = lambda i: (0, 0)
    return pl.pallas_call(
        _proj_body,
        grid=(GRID,),
        in_specs=[
            pl.BlockSpec((R, D), blk),
            pl.BlockSpec((2 * D, D), full),
            pl.BlockSpec((1, D), full),
        ],
        out_specs=[pl.BlockSpec((R, D), blk), pl.BlockSpec((R, D), blk)],
        out_shape=[
            jax.ShapeDtypeStruct((NP, D), jnp.float32),
            jax.ShapeDtypeStruct((NP, D), jnp.float32),
        ],
    )(h, Mw, Mb.reshape(1, D))


def _layer_body(h_ref, s_ref, q_ref, mx_ref, mn_ref, deg_ref, b_ref,
                uw_ref, ub_ref, mixw_ref, mixb_ref, out_ref, *, last):
    h = h_ref[...]
    deg = deg_ref[...]
    B = b_ref[...]
    S = s_ref[...]
    cnt = jnp.maximum(deg, 1.0)
    pos = deg > 0.0
    mean = (S + deg * B) / cnt
    mx = jnp.where(pos, mx_ref[...] + B, 0.0)
    mn = jnp.where(pos, mn_ref[...] + B, 0.0)
    sq = (q_ref[...] + 2.0 * B * S + deg * B * B) / cnt
    var = jnp.maximum(sq - mean * mean, 0.0)
    std = jnp.sqrt(var + 1e-5)
    logd = jnp.log(deg + 1.0)
    s_amp = logd * (1.0 / DELTA)
    s_att = DELTA / jnp.maximum(logd, 1e-6)
    pieces = [h, mean, mx, mn, std,
              mean * s_amp, mx * s_amp, mn * s_amp, std * s_amp,
              mean * s_att, mx * s_att, mn * s_att, std * s_att]
    u = ub_ref[...]
    for k, p in enumerate(pieces):
        u = u + jnp.dot(p, uw_ref[k * D:(k + 1) * D, :],
                        preferred_element_type=jnp.float32)
    t = jnp.dot(u, mixw_ref[...], preferred_element_type=jnp.float32) + mixb_ref[...]
    t = jnp.where(t > 0, t, 0.01 * t)
    t = h + t
    if not last:
        t = jnp.maximum(t, 0.0)
    out_ref[...] = t


def _tc_layer(h, S, Q, MX, MN, degb, B, Uw, Ub, MixW, Mixb, last):
    blk = lambda i: (i, 0)
    full = lambda i: (0, 0)
    return pl.pallas_call(
        functools.partial(_layer_body, last=last),
        grid=(GRID,),
        in_specs=[pl.BlockSpec((R, D), blk)] * 7 + [
            pl.BlockSpec((13 * D, D), full),
            pl.BlockSpec((1, D), full),
            pl.BlockSpec((D, D), full),
            pl.BlockSpec((1, D), full),
        ],
        out_specs=pl.BlockSpec((R, D), blk),
        out_shape=jax.ShapeDtypeStruct((NP, D), jnp.float32),
    )(h, S, Q, MX, MN, degb, B, Uw, Ub.reshape(1, D), MixW, Mixb.reshape(1, D))


def _fc_body(h_ref, w_ref, b_ref, o_ref):
    o_ref[...] = jnp.dot(h_ref[...], w_ref[...],
                         preferred_element_type=jnp.float32) + b_ref[...]


def _tc_fc(h, FCw_pad, FCb_pad):
    blk = lambda i: (i, 0)
    full = lambda i: (0, 0)
    return pl.pallas_call(
        _fc_body,
        grid=(GRID,),
        in_specs=[
            pl.BlockSpec((R, D), blk),
            pl.BlockSpec((D, D), full),
            pl.BlockSpec((1, D), full),
        ],
        out_specs=pl.BlockSpec((R, D), blk),
        out_shape=jax.ShapeDtypeStruct((NP, D), jnp.float32),
    )(h, FCw_pad, FCb_pad)


def _segment_stats(A, src, dst, n):
    S = jax.ops.segment_sum(A[src], dst, num_segments=n)
    Q = jax.ops.segment_sum(A[src] * A[src], dst, num_segments=n)
    MX = jax.ops.segment_max(A[src], dst, num_segments=n)
    MN = jax.ops.segment_min(A[src], dst, num_segments=n)
    return S, Q, MX, MN


def kernel(in_feat, edge_index, Mw0, Mb0, Uw0, Ub0, MixW0, Mixb0,
           Mw1, Mb1, Uw1, Ub1, MixW1, Mixb1, Mw2, Mb2, Uw2, Ub2, MixW2, Mixb2,
           FCw, FCb):
    n = in_feat.shape[0]
    src = edge_index[0]
    dst = edge_index[1]
    h = jnp.pad(in_feat, ((0, NP - n), (0, 0)))

    deg = jax.ops.segment_sum(jnp.ones(src.shape, jnp.float32), dst, num_segments=n)
    degb = jnp.pad(deg, (0, NP - n))[:, None] * jnp.ones((1, D), jnp.float32)

    layers = [(Mw0, Mb0, Uw0, Ub0, MixW0, Mixb0),
              (Mw1, Mb1, Uw1, Ub1, MixW1, Mixb1),
              (Mw2, Mb2, Uw2, Ub2, MixW2, Mixb2)]
    for l, (Mw, Mb, Uw, Ub, MixW, Mixb) in enumerate(layers):
        A, B = _tc_proj(h, Mw, Mb)
        S, Q, MX, MN = _segment_stats(A[:n], src, dst, n)
        pad4 = lambda x: jnp.pad(x, ((0, NP - n), (0, 0)))
        h = _tc_layer(h, pad4(S), pad4(Q), pad4(MX), pad4(MN), degb, B,
                      Uw, Ub, MixW, Mixb, last=(l == 2))

    FCw_pad = jnp.pad(FCw, ((0, 0), (0, D - FCw.shape[1])))
    FCb_pad = jnp.pad(FCb, (0, D - FCb.shape[0]))
    out = _tc_fc(h, FCw_pad, FCb_pad.reshape(1, D))
    return out[:n, :FCw.shape[1]]


# speedup vs baseline: 2.5550x; 2.2159x over previous
import functools
import jax
import jax.numpy as jnp
from jax import lax
from jax.experimental import pallas as pl
from jax.experimental.pallas import tpu as pltpu
from jax.experimental.pallas import tpu_sc as plsc

D = 128
DELTA = 3.5
NP = 10240
R = 1024
GRID = NP // R


def _proj_body(h_ref, mw_ref, mb_ref, a_ref, b_ref):
    h = h_ref[...]
    a_ref[...] = jnp.dot(h, mw_ref[0:D, :], preferred_element_type=jnp.float32)
    b_ref[...] = jnp.dot(h, mw_ref[D:2 * D, :], preferred_element_type=jnp.float32) + mb_ref[...]


def _tc_proj(h, Mw, Mb):
    blk = lambda i: (i, 0)
    full = lambda i: (0, 0)
    return pl.pallas_call(
        _proj_body,
        grid=(GRID,),
        in_specs=[
            pl.BlockSpec((R, D), blk),
            pl.BlockSpec((2 * D, D), full),
            pl.BlockSpec((1, D), full),
        ],
        out_specs=[pl.BlockSpec((R, D), blk), pl.BlockSpec((R, D), blk)],
        out_shape=[
            jax.ShapeDtypeStruct((NP, D), jnp.float32),
            jax.ShapeDtypeStruct((NP, D), jnp.float32),
        ],
    )(h, Mw, Mb.reshape(1, D))


def _layer_body(h_ref, s_ref, q_ref, mx_ref, mn_ref, deg_ref, b_ref,
                uw_ref, ub_ref, mixw_ref, mixb_ref, out_ref, *, last):
    h = h_ref[...]
    deg = deg_ref[...]
    B = b_ref[...]
    S = s_ref[...]
    cnt = jnp.maximum(deg, 1.0)
    pos = deg > 0.0
    mean = (S + deg * B) / cnt
    mx = jnp.where(pos, mx_ref[...] + B, 0.0)
    mn = jnp.where(pos, mn_ref[...] + B, 0.0)
    sq = (q_ref[...] + 2.0 * B * S + deg * B * B) / cnt
    var = jnp.maximum(sq - mean * mean, 0.0)
    std = jnp.sqrt(var + 1e-5)
    logd = jnp.log(deg + 1.0)
    s_amp = logd * (1.0 / DELTA)
    s_att = DELTA / jnp.maximum(logd, 1e-6)
    pieces = [h, mean, mx, mn, std,
              mean * s_amp, mx * s_amp, mn * s_amp, std * s_amp,
              mean * s_att, mx * s_att, mn * s_att, std * s_att]
    u = ub_ref[...]
    for k, p in enumerate(pieces):
        u = u + jnp.dot(p, uw_ref[k * D:(k + 1) * D, :],
                        preferred_element_type=jnp.float32)
    t = jnp.dot(u, mixw_ref[...], preferred_element_type=jnp.float32) + mixb_ref[...]
    t = jnp.where(t > 0, t, 0.01 * t)
    t = h + t
    if not last:
        t = jnp.maximum(t, 0.0)
    out_ref[...] = t


def _tc_layer(h, S, Q, MX, MN, degb, B, Uw, Ub, MixW, Mixb, last):
    blk = lambda i: (i, 0)
    full = lambda i: (0, 0)
    return pl.pallas_call(
        functools.partial(_layer_body, last=last),
        grid=(GRID,),
        in_specs=[pl.BlockSpec((R, D), blk)] * 7 + [
            pl.BlockSpec((13 * D, D), full),
            pl.BlockSpec((1, D), full),
            pl.BlockSpec((D, D), full),
            pl.BlockSpec((1, D), full),
        ],
        out_specs=pl.BlockSpec((R, D), blk),
        out_shape=jax.ShapeDtypeStruct((NP, D), jnp.float32),
    )(h, S, Q, MX, MN, degb, B, Uw, Ub.reshape(1, D), MixW, Mixb.reshape(1, D))


def _fc_body(h_ref, w_ref, b_ref, o_ref):
    o_ref[...] = jnp.dot(h_ref[...], w_ref[...],
                         preferred_element_type=jnp.float32) + b_ref[...]


def _tc_fc(h, FCw_pad, FCb_pad):
    blk = lambda i: (i, 0)
    full = lambda i: (0, 0)
    return pl.pallas_call(
        _fc_body,
        grid=(GRID,),
        in_specs=[
            pl.BlockSpec((R, D), blk),
            pl.BlockSpec((D, D), full),
            pl.BlockSpec((1, D), full),
        ],
        out_specs=pl.BlockSpec((R, D), blk),
        out_shape=jax.ShapeDtypeStruct((NP, D), jnp.float32),
    )(h, FCw_pad, FCb_pad)


NCORES = 2
NSUB = 16
NWORK = NCORES * NSUB
NBPT = 2
NBUCK = NWORK * NBPT
BKT = NP // NBUCK
CAP = 2048
GB = 128
CH = 1280
FINIT = 3.0e38


def _sc_stats_body(a_hbm, src_hbm, dst_hbm, s_out, q_out, mx_out, mn_out,
                   deg_out, sbuf, dbuf, ring_s, ring_d, rows,
                   S, Q, MX, MN, degacc, degbuf, sem):
    cid = lax.axis_index("c")
    sid = lax.axis_index("s")
    wid = sid * NCORES + cid
    iota = lax.iota(jnp.int32, 16)
    assert src_hbm.shape[0] % CH == 0 and CH % 16 == 0
    nchunk = src_hbm.shape[0] // CH

    def drain_batch(dr):
        base = pl.multiple_of(dr & (CAP - 1), GB)
        pltpu.async_copy(a_hbm.at[ring_s.at[pl.ds(base, GB)]], rows, sem).wait()
        ones = jnp.ones((16,), jnp.float32)

        def group(g, _):
            gb = pl.multiple_of(g * 16, 16)
            dlv = ring_d[pl.ds(base + gb, 16)]
            for i in range(16):
                dl = dlv[i]
                plsc.addupdate(degacc.at[dl, :], ones)
                for f in range(8):
                    sl = pl.ds(16 * f, 16)
                    r = rows[gb + i, sl]
                    plsc.addupdate(S.at[dl, sl], r)
                    plsc.addupdate(Q.at[dl, sl], r * r)
                    MX[dl, sl] = jnp.maximum(MX[dl, sl], r)
                    MN[dl, sl] = jnp.minimum(MN[dl, sl], r)
            return 0

        lax.fori_loop(0, GB // 16, group, 0)

    def run_bucket(b):
        lo = b * BKT
        zeros = jnp.zeros((16,), jnp.float32)
        neg = jnp.full((16,), -FINIT, jnp.float32)
        pos = jnp.full((16,), FINIT, jnp.float32)

        def initr(r, _):
            for f in range(8):
                sl = pl.ds(16 * f, 16)
                S[r, sl] = zeros
                Q[r, sl] = zeros
                MX[r, sl] = neg
                MN[r, sl] = pos
            degacc[r, :] = zeros
            return 0

        lax.fori_loop(0, BKT + 1, initr, 0)

        def chunk(c, carry):
            off, dr = carry
            cb = pl.multiple_of(c * CH, 8)
            pltpu.sync_copy(src_hbm.at[pl.ds(cb, CH)], sbuf)
            pltpu.sync_copy(dst_hbm.at[pl.ds(cb, CH)], dbuf)

            def scan16(j, off):
                sl = pl.ds(pl.multiple_of(j * 16, 16), 16)
                dvec = dbuf[sl]
                svec = sbuf[sl]
                mask = (dvec >= lo) & (dvec < lo + BKT)
                mi = jnp.where(mask, jnp.ones((16,), jnp.int32),
                               jnp.zeros((16,), jnp.int32))
                idx = (off + plsc.cumsum(mi) - 1) & (CAP - 1)
                plsc.store_scatter(ring_s, [idx], svec, mask=mask)
                plsc.store_scatter(ring_d, [idx], dvec - lo, mask=mask)
                return off + jnp.sum(mi)

            off = lax.fori_loop(0, CH // 16, scan16, off)

            def wbody(d):
                drain_batch(d)
                return d + GB

            dr = lax.while_loop(lambda d: off - d >= GB, wbody, dr)
            return (off, dr)

        off, dr = lax.fori_loop(0, nchunk, chunk,
                                (jnp.int32(0), jnp.int32(0)))

        @pl.when(off > dr)
        def _():
            zi = jnp.zeros((16,), jnp.int32)
            for k in range(8):
                idxp = (off + k * 16 + iota) & (CAP - 1)
                plsc.store_scatter(ring_s, [idxp], zi + wid)
                plsc.store_scatter(ring_d, [idxp], zi + BKT)
            drain_batch(dr)

        lob = pl.multiple_of(lo, 8)
        pltpu.sync_copy(S.at[pl.ds(0, BKT), :], s_out.at[pl.ds(lob, BKT), :])
        pltpu.sync_copy(Q.at[pl.ds(0, BKT), :], q_out.at[pl.ds(lob, BKT), :])
        pltpu.sync_copy(MX.at[pl.ds(0, BKT), :], mx_out.at[pl.ds(lob, BKT), :])
        pltpu.sync_copy(MN.at[pl.ds(0, BKT), :], mn_out.at[pl.ds(lob, BKT), :])

        def degg(g, _):
            v = plsc.load_gather(degacc, [g * 16 + iota, jnp.zeros((16,), jnp.int32)])
            degbuf[pl.ds(pl.multiple_of(g * 16, 16), 16)] = v
            return 0

        lax.fori_loop(0, BKT // 16, degg, 0)
        pltpu.sync_copy(degbuf, deg_out.at[pl.ds(lob, BKT)])

    for bi in range(NBPT):
        run_bucket(wid * NBPT + bi)


def _sc_segment_stats(A, src, dst):
    mesh = plsc.VectorSubcoreMesh(core_axis_name="c", subcore_axis_name="s",
                                  num_cores=NCORES, num_subcores=NSUB)
    f32 = jnp.float32
    fn = pl.kernel(
        _sc_stats_body,
        out_type=[jax.ShapeDtypeStruct((NP, D), f32)] * 4
        + [jax.ShapeDtypeStruct((NP,), f32)],
        mesh=mesh,
        scratch_types=[
            pltpu.VMEM((CH,), jnp.int32),
            pltpu.VMEM((CH,), jnp.int32),
            pltpu.VMEM((CAP,), jnp.int32),
            pltpu.VMEM((CAP,), jnp.int32),
            pltpu.VMEM((GB, D), f32),
            pltpu.VMEM((BKT + 1, D), f32),
            pltpu.VMEM((BKT + 1, D), f32),
            pltpu.VMEM((BKT + 1, D), f32),
            pltpu.VMEM((BKT + 1, D), f32),
            pltpu.VMEM((BKT + 1, 16), f32),
            pltpu.VMEM((BKT,), f32),
            pltpu.SemaphoreType.DMA,
        ],
        compiler_params=pltpu.CompilerParams(needs_layout_passes=False),
    )
    return fn(A, src, dst)


def kernel(in_feat, edge_index, Mw0, Mb0, Uw0, Ub0, MixW0, Mixb0,
           Mw1, Mb1, Uw1, Ub1, MixW1, Mixb1, Mw2, Mb2, Uw2, Ub2, MixW2, Mixb2,
           FCw, FCb):
    n = in_feat.shape[0]
    src = edge_index[0]
    dst = edge_index[1]
    h = jnp.pad(in_feat, ((0, NP - n), (0, 0)))

    layers = [(Mw0, Mb0, Uw0, Ub0, MixW0, Mixb0),
              (Mw1, Mb1, Uw1, Ub1, MixW1, Mixb1),
              (Mw2, Mb2, Uw2, Ub2, MixW2, Mixb2)]
    degb = None
    for l, (Mw, Mb, Uw, Ub, MixW, Mixb) in enumerate(layers):
        A, B = _tc_proj(h, Mw, Mb)
        S, Q, MX, MN, deg = _sc_segment_stats(A, src, dst)
        if degb is None:
            degb = deg[:, None] * jnp.ones((1, D), jnp.float32)
        h = _tc_layer(h, S, Q, MX, MN, degb, B,
                      Uw, Ub, MixW, Mixb, last=(l == 2))

    FCw_pad = jnp.pad(FCw, ((0, 0), (0, D - FCw.shape[1])))
    FCb_pad = jnp.pad(FCb, (0, D - FCb.shape[0]))
    out = _tc_fc(h, FCw_pad, FCb_pad.reshape(1, D))
    return out[:n, :FCw.shape[1]]


# speedup vs baseline: 4.1530x; 1.6254x over previous
import functools
import jax
import jax.numpy as jnp
from jax import lax
from jax.experimental import pallas as pl
from jax.experimental.pallas import tpu as pltpu
from jax.experimental.pallas import tpu_sc as plsc

D = 128
DELTA = 3.5
NP = 10240
R = 1024
GRID = NP // R


def _proj_body(h_ref, mw_ref, mb_ref, a_ref, b_ref):
    h = h_ref[...]
    a_ref[...] = jnp.dot(h, mw_ref[0:D, :], preferred_element_type=jnp.float32)
    b_ref[...] = jnp.dot(h, mw_ref[D:2 * D, :], preferred_element_type=jnp.float32) + mb_ref[...]


def _tc_proj(h, Mw, Mb):
    blk = lambda i: (i, 0)
    full = lambda i: (0, 0)
    return pl.pallas_call(
        _proj_body,
        grid=(GRID,),
        in_specs=[
            pl.BlockSpec((R, D), blk),
            pl.BlockSpec((2 * D, D), full),
            pl.BlockSpec((1, D), full),
        ],
        out_specs=[pl.BlockSpec((R, D), blk), pl.BlockSpec((R, D), blk)],
        out_shape=[
            jax.ShapeDtypeStruct((NP, D), jnp.float32),
            jax.ShapeDtypeStruct((NP, D), jnp.float32),
        ],
    )(h, Mw, Mb.reshape(1, D))


def _layer_body(h_ref, s_ref, q_ref, mx_ref, mn_ref, deg_ref, b_ref,
                uw_ref, ub_ref, mixw_ref, mixb_ref, out_ref, *, last):
    h = h_ref[...]
    deg = deg_ref[...]
    B = b_ref[...]
    S = s_ref[...]
    cnt = jnp.maximum(deg, 1.0)
    pos = deg > 0.0
    mean = (S + deg * B) / cnt
    mx = jnp.where(pos, mx_ref[...] + B, 0.0)
    mn = jnp.where(pos, mn_ref[...] + B, 0.0)
    sq = (q_ref[...] + 2.0 * B * S + deg * B * B) / cnt
    var = jnp.maximum(sq - mean * mean, 0.0)
    std = jnp.sqrt(var + 1e-5)
    logd = jnp.log(deg + 1.0)
    s_amp = logd * (1.0 / DELTA)
    s_att = DELTA / jnp.maximum(logd, 1e-6)
    pieces = [h, mean, mx, mn, std,
              mean * s_amp, mx * s_amp, mn * s_amp, std * s_amp,
              mean * s_att, mx * s_att, mn * s_att, std * s_att]
    u = ub_ref[...]
    for k, p in enumerate(pieces):
        u = u + jnp.dot(p, uw_ref[k * D:(k + 1) * D, :],
                        preferred_element_type=jnp.float32)
    t = jnp.dot(u, mixw_ref[...], preferred_element_type=jnp.float32) + mixb_ref[...]
    t = jnp.where(t > 0, t, 0.01 * t)
    t = h + t
    if not last:
        t = jnp.maximum(t, 0.0)
    out_ref[...] = t


def _tc_layer(h, S, Q, MX, MN, degb, B, Uw, Ub, MixW, Mixb, last):
    blk = lambda i: (i, 0)
    full = lambda i: (0, 0)
    return pl.pallas_call(
        functools.partial(_layer_body, last=last),
        grid=(GRID,),
        in_specs=[pl.BlockSpec((R, D), blk)] * 7 + [
            pl.BlockSpec((13 * D, D), full),
            pl.BlockSpec((1, D), full),
            pl.BlockSpec((D, D), full),
            pl.BlockSpec((1, D), full),
        ],
        out_specs=pl.BlockSpec((R, D), blk),
        out_shape=jax.ShapeDtypeStruct((NP, D), jnp.float32),
    )(h, S, Q, MX, MN, degb, B, Uw, Ub.reshape(1, D), MixW, Mixb.reshape(1, D))


def _fc_body(h_ref, w_ref, b_ref, o_ref):
    o_ref[...] = jnp.dot(h_ref[...], w_ref[...],
                         preferred_element_type=jnp.float32) + b_ref[...]


def _tc_fc(h, FCw_pad, FCb_pad):
    blk = lambda i: (i, 0)
    full = lambda i: (0, 0)
    return pl.pallas_call(
        _fc_body,
        grid=(GRID,),
        in_specs=[
            pl.BlockSpec((R, D), blk),
            pl.BlockSpec((D, D), full),
            pl.BlockSpec((1, D), full),
        ],
        out_specs=pl.BlockSpec((R, D), blk),
        out_shape=jax.ShapeDtypeStruct((NP, D), jnp.float32),
    )(h, FCw_pad, FCb_pad)


NCORES = 2
NSUB = 16
NWORK = NCORES * NSUB
NBPT = 2
NBUCK = NWORK * NBPT
BKT = NP // NBUCK
CAP = 2048
GB = 128
CH = 1280
CH2 = 2048
EPB = 320000 + 4096
FINIT = 3.0e38


def _sc_part_body(src_hbm, dst_hbm, packed_out, cnt_out, sbuf, dbuf, ring, cbuf):
    cid = lax.axis_index("c")
    sid = lax.axis_index("s")
    wid = sid * NCORES + cid
    iota = lax.iota(jnp.int32, 16)
    assert src_hbm.shape[0] % CH == 0 and CH % 16 == 0
    nchunk = src_hbm.shape[0] // CH
    ones_i = jnp.ones((16,), jnp.int32)
    zeros_i = jnp.zeros((16,), jnp.int32)

    def run_bucket(b):
        lo = b * BKT
        pbase = b * EPB

        def flush(d):
            base = pl.multiple_of(d & (CAP - 1), GB)
            pltpu.sync_copy(ring.at[pl.ds(base, GB)],
                            packed_out.at[pl.ds(pl.multiple_of(pbase + d, 8), GB)])
            return d + GB

        def chunk(c, carry):
            off, dr = carry
            cb = pl.multiple_of(c * CH, 8)
            pltpu.sync_copy(src_hbm.at[pl.ds(cb, CH)], sbuf)
            pltpu.sync_copy(dst_hbm.at[pl.ds(cb, CH)], dbuf)

            def scan16(j, off):
                sl = pl.ds(pl.multiple_of(j * 16, 16), 16)
                dvec = dbuf[sl]
                svec = sbuf[sl]
                mask = (dvec >= lo) & (dvec < lo + BKT)
                mi = jnp.where(mask, ones_i, zeros_i)
                idx = (off + plsc.cumsum(mi) - 1) & (CAP - 1)
                w = svec | ((dvec - lo) << 16)
                plsc.store_scatter(ring, [idx], w, mask=mask)
                return off + jnp.sum(mi)

            off = lax.fori_loop(0, CH // 16, scan16, off)
            dr = lax.while_loop(lambda d: off - d >= GB, flush, dr)
            return (off, dr)

        off, dr = lax.fori_loop(0, nchunk, chunk,
                                (jnp.int32(0), jnp.int32(0)))

        padv = zeros_i + jnp.int32(BKT << 16)
        for k in range(GB // 16):
            idxp = (off + k * 16 + iota) & (CAP - 1)
            plsc.store_scatter(ring, [idxp], padv)
        W = (off + GB - 1) & jnp.int32(-GB)
        dr = lax.while_loop(lambda d: d < W, flush, dr)
        cbuf[...] = zeros_i + W
        pltpu.sync_copy(cbuf, cnt_out.at[pl.ds(pl.multiple_of(b * 16, 16), 16)])

    for bi in range(NBPT):
        run_bucket(wid * NBPT + bi)


def _sc_partition(src, dst):
    mesh = plsc.VectorSubcoreMesh(core_axis_name="c", subcore_axis_name="s",
                                  num_cores=NCORES, num_subcores=NSUB)
    fn = pl.kernel(
        _sc_part_body,
        out_type=[jax.ShapeDtypeStruct((NBUCK * EPB,), jnp.int32),
                  jax.ShapeDtypeStruct((NBUCK * 16,), jnp.int32)],
        mesh=mesh,
        scratch_types=[
            pltpu.VMEM((CH,), jnp.int32),
            pltpu.VMEM((CH,), jnp.int32),
            pltpu.VMEM((CAP,), jnp.int32),
            pltpu.VMEM((16,), jnp.int32),
        ],
        compiler_params=pltpu.CompilerParams(needs_layout_passes=False),
    )
    return fn(src, dst)


def _sc_stats_body(a_hbm, packed_hbm, cnt_hbm, s_out, q_out, mx_out, mn_out,
                   deg_out, pbuf, sidx, rows,
                   S, Q, MX, MN, degacc, degbuf, cbuf, sem):
    cid = lax.axis_index("c")
    sid = lax.axis_index("s")
    wid = sid * NCORES + cid
    iota = lax.iota(jnp.int32, 16)
    ones = jnp.ones((16,), jnp.float32)
    mask16 = jnp.int32(0xFFFF)

    def run_bucket(b):
        lo = b * BKT
        pbase = b * EPB
        zeros = jnp.zeros((16,), jnp.float32)
        neg = jnp.full((16,), -FINIT, jnp.float32)
        pos = jnp.full((16,), FINIT, jnp.float32)

        def initr(r, _):
            for f in range(8):
                sl = pl.ds(16 * f, 16)
                S[r, sl] = zeros
                Q[r, sl] = zeros
                MX[r, sl] = neg
                MN[r, sl] = pos
            degacc[r, :] = zeros
            return 0

        lax.fori_loop(0, BKT + 1, initr, 0)

        pltpu.sync_copy(cnt_hbm.at[pl.ds(pl.multiple_of(b * 16, 16), 16)], cbuf)
        W = cbuf[pl.ds(0, 16)][0]

        def chunk(c, _):
            cb = c * CH2
            pltpu.sync_copy(packed_hbm.at[pl.ds(pl.multiple_of(pbase + cb, 8), CH2)], pbuf)
            nb = jnp.minimum((W - cb) // GB, CH2 // GB)

            def batch(k, _):
                kb = pl.multiple_of(k * GB, GB)
                for g in range(GB // 16):
                    wv = pbuf[pl.ds(pl.multiple_of(kb + g * 16, 16), 16)]
                    sidx[pl.ds(g * 16, 16)] = wv & mask16
                pltpu.async_copy(a_hbm.at[sidx], rows, sem).wait()

                def group(g, _):
                    gb = pl.multiple_of(g * 16, 16)
                    wv = pbuf[pl.ds(pl.multiple_of(kb + gb, 16), 16)]
                    dlv = wv >> 16
                    for i in range(16):
                        dl = dlv[i]
                        plsc.addupdate(degacc.at[dl, :], ones)
                        for f in range(8):
                            sl = pl.ds(16 * f, 16)
                            r = rows[gb + i, sl]
                            plsc.addupdate(S.at[dl, sl], r)
                            plsc.addupdate(Q.at[dl, sl], r * r)
                            MX[dl, sl] = jnp.maximum(MX[dl, sl], r)
                            MN[dl, sl] = jnp.minimum(MN[dl, sl], r)
                    return 0

                lax.fori_loop(0, GB // 16, group, 0)
                return 0

            lax.fori_loop(0, nb, batch, 0)
            return 0

        nchunk = (W + CH2 - 1) // CH2
        lax.fori_loop(0, nchunk, chunk, 0)

        lob = pl.multiple_of(lo, 8)
        pltpu.sync_copy(S.at[pl.ds(0, BKT), :], s_out.at[pl.ds(lob, BKT), :])
        pltpu.sync_copy(Q.at[pl.ds(0, BKT), :], q_out.at[pl.ds(lob, BKT), :])
        pltpu.sync_copy(MX.at[pl.ds(0, BKT), :], mx_out.at[pl.ds(lob, BKT), :])
        pltpu.sync_copy(MN.at[pl.ds(0, BKT), :], mn_out.at[pl.ds(lob, BKT), :])

        def degg(g, _):
            v = plsc.load_gather(degacc, [g * 16 + iota, jnp.zeros((16,), jnp.int32)])
            degbuf[pl.ds(pl.multiple_of(g * 16, 16), 16)] = v
            return 0

        lax.fori_loop(0, BKT // 16, degg, 0)
        pltpu.sync_copy(degbuf, deg_out.at[pl.ds(lob, BKT)])

    for bi in range(NBPT):
        run_bucket(wid * NBPT + bi)


def _sc_segment_stats(A, packed, cnts):
    mesh = plsc.VectorSubcoreMesh(core_axis_name="c", subcore_axis_name="s",
                                  num_cores=NCORES, num_subcores=NSUB)
    f32 = jnp.float32
    fn = pl.kernel(
        _sc_stats_body,
        out_type=[jax.ShapeDtypeStruct((NP, D), f32)] * 4
        + [jax.ShapeDtypeStruct((NP,), f32)],
        mesh=mesh,
        scratch_types=[
            pltpu.VMEM((CH2,), jnp.int32),
            pltpu.VMEM((GB,), jnp.int32),
            pltpu.VMEM((GB, D), f32),
            pltpu.VMEM((BKT + 1, D), f32),
            pltpu.VMEM((BKT + 1, D), f32),
            pltpu.VMEM((BKT + 1, D), f32),
            pltpu.VMEM((BKT + 1, D), f32),
            pltpu.VMEM((BKT + 1, 16), f32),
            pltpu.VMEM((BKT,), f32),
            pltpu.VMEM((16,), jnp.int32),
            pltpu.SemaphoreType.DMA,
        ],
        compiler_params=pltpu.CompilerParams(needs_layout_passes=False),
    )
    return fn(A, packed, cnts)


def kernel(in_feat, edge_index, Mw0, Mb0, Uw0, Ub0, MixW0, Mixb0,
           Mw1, Mb1, Uw1, Ub1, MixW1, Mixb1, Mw2, Mb2, Uw2, Ub2, MixW2, Mixb2,
           FCw, FCb):
    n = in_feat.shape[0]
    src = edge_index[0]
    dst = edge_index[1]
    h = jnp.pad(in_feat, ((0, NP - n), (0, 0)))

    packed, cnts = _sc_partition(src, dst)

    layers = [(Mw0, Mb0, Uw0, Ub0, MixW0, Mixb0),
              (Mw1, Mb1, Uw1, Ub1, MixW1, Mixb1),
              (Mw2, Mb2, Uw2, Ub2, MixW2, Mixb2)]
    degb = None
    for l, (Mw, Mb, Uw, Ub, MixW, Mixb) in enumerate(layers):
        A, B = _tc_proj(h, Mw, Mb)
        S, Q, MX, MN, deg = _sc_segment_stats(A, packed, cnts)
        if degb is None:
            degb = deg[:, None] * jnp.ones((1, D), jnp.float32)
        h = _tc_layer(h, S, Q, MX, MN, degb, B,
                      Uw, Ub, MixW, Mixb, last=(l == 2))

    FCw_pad = jnp.pad(FCw, ((0, 0), (0, D - FCw.shape[1])))
    FCb_pad = jnp.pad(FCb, (0, D - FCb.shape[0]))
    out = _tc_fc(h, FCw_pad, FCb_pad.reshape(1, D))
    return out[:n, :FCw.shape[1]]


# speedup vs baseline: 4.4150x; 1.0631x over previous
import functools
import jax
import jax.numpy as jnp
from jax import lax
from jax.experimental import pallas as pl
from jax.experimental.pallas import tpu as pltpu
from jax.experimental.pallas import tpu_sc as plsc

D = 128
DELTA = 3.5
NP = 10240
R = 1024
GRID = NP // R


def _proj_body(h_ref, mw_ref, mb_ref, a_ref, b_ref):
    h = h_ref[...]
    a_ref[...] = jnp.dot(h, mw_ref[0:D, :], preferred_element_type=jnp.float32)
    b_ref[...] = jnp.dot(h, mw_ref[D:2 * D, :], preferred_element_type=jnp.float32) + mb_ref[...]


def _tc_proj(h, Mw, Mb):
    blk = lambda i: (i, 0)
    full = lambda i: (0, 0)
    return pl.pallas_call(
        _proj_body,
        grid=(GRID,),
        in_specs=[
            pl.BlockSpec((R, D), blk),
            pl.BlockSpec((2 * D, D), full),
            pl.BlockSpec((1, D), full),
        ],
        out_specs=[pl.BlockSpec((R, D), blk), pl.BlockSpec((R, D), blk)],
        out_shape=[
            jax.ShapeDtypeStruct((NP, D), jnp.float32),
            jax.ShapeDtypeStruct((NP, D), jnp.float32),
        ],
    )(h, Mw, Mb.reshape(1, D))


def _layer_body(h_ref, s_ref, q_ref, mx_ref, mn_ref, deg_ref, b_ref,
                uw_ref, ub_ref, mixw_ref, mixb_ref, out_ref, *, last):
    h = h_ref[...]
    deg = deg_ref[...]
    B = b_ref[...]
    S = s_ref[...]
    cnt = jnp.maximum(deg, 1.0)
    pos = deg > 0.0
    mean = (S + deg * B) / cnt
    mx = jnp.where(pos, mx_ref[...] + B, 0.0)
    mn = jnp.where(pos, mn_ref[...] + B, 0.0)
    sq = (q_ref[...] + 2.0 * B * S + deg * B * B) / cnt
    var = jnp.maximum(sq - mean * mean, 0.0)
    std = jnp.sqrt(var + 1e-5)
    logd = jnp.log(deg + 1.0)
    s_amp = logd * (1.0 / DELTA)
    s_att = DELTA / jnp.maximum(logd, 1e-6)
    pieces = [h, mean, mx, mn, std,
              mean * s_amp, mx * s_amp, mn * s_amp, std * s_amp,
              mean * s_att, mx * s_att, mn * s_att, std * s_att]
    u = ub_ref[...]
    for k, p in enumerate(pieces):
        u = u + jnp.dot(p, uw_ref[k * D:(k + 1) * D, :],
                        preferred_element_type=jnp.float32)
    t = jnp.dot(u, mixw_ref[...], preferred_element_type=jnp.float32) + mixb_ref[...]
    t = jnp.where(t > 0, t, 0.01 * t)
    t = h + t
    if not last:
        t = jnp.maximum(t, 0.0)
    out_ref[...] = t


def _tc_layer(h, S, Q, MX, MN, degb, B, Uw, Ub, MixW, Mixb, last):
    blk = lambda i: (i, 0)
    full = lambda i: (0, 0)
    return pl.pallas_call(
        functools.partial(_layer_body, last=last),
        grid=(GRID,),
        in_specs=[pl.BlockSpec((R, D), blk)] * 7 + [
            pl.BlockSpec((13 * D, D), full),
            pl.BlockSpec((1, D), full),
            pl.BlockSpec((D, D), full),
            pl.BlockSpec((1, D), full),
        ],
        out_specs=pl.BlockSpec((R, D), blk),
        out_shape=jax.ShapeDtypeStruct((NP, D), jnp.float32),
    )(h, S, Q, MX, MN, degb, B, Uw, Ub.reshape(1, D), MixW, Mixb.reshape(1, D))


def _fc_body(h_ref, w_ref, b_ref, o_ref):
    o_ref[...] = jnp.dot(h_ref[...], w_ref[...],
                         preferred_element_type=jnp.float32) + b_ref[...]


def _tc_fc(h, FCw_pad, FCb_pad):
    blk = lambda i: (i, 0)
    full = lambda i: (0, 0)
    return pl.pallas_call(
        _fc_body,
        grid=(GRID,),
        in_specs=[
            pl.BlockSpec((R, D), blk),
            pl.BlockSpec((D, D), full),
            pl.BlockSpec((1, D), full),
        ],
        out_specs=pl.BlockSpec((R, D), blk),
        out_shape=jax.ShapeDtypeStruct((NP, D), jnp.float32),
    )(h, FCw_pad, FCb_pad)


NCORES = 2
NSUB = 16
NWORK = NCORES * NSUB
NBPT = 2
NBUCK = NWORK * NBPT
BKT = NP // NBUCK
CAP = 2048
RB = 128
GB = 64
CH = 1280
CH2 = 2048
EPB = 320000 + 4096
FINIT = 3.0e38


def _sc_part_body(src_hbm, dst_hbm, packed_out, cnt_out, sbuf, dbuf, ring, cbuf):
    cid = lax.axis_index("c")
    sid = lax.axis_index("s")
    wid = sid * NCORES + cid
    iota = lax.iota(jnp.int32, 16)
    assert src_hbm.shape[0] % CH == 0 and CH % 16 == 0
    nchunk = src_hbm.shape[0] // CH
    ones_i = jnp.ones((16,), jnp.int32)
    zeros_i = jnp.zeros((16,), jnp.int32)

    def run_bucket(b):
        lo = b * BKT
        pbase = b * EPB

        def flush(d):
            base = pl.multiple_of(d & (CAP - 1), RB)
            pltpu.sync_copy(ring.at[pl.ds(base, RB)],
                            packed_out.at[pl.ds(pl.multiple_of(pbase + d, 8), RB)])
            return d + RB

        def chunk(c, carry):
            off, dr = carry
            cb = pl.multiple_of(c * CH, 8)
            pltpu.sync_copy(src_hbm.at[pl.ds(cb, CH)], sbuf)
            pltpu.sync_copy(dst_hbm.at[pl.ds(cb, CH)], dbuf)

            def scan16(j, off):
                sl = pl.ds(pl.multiple_of(j * 16, 16), 16)
                dvec = dbuf[sl]
                svec = sbuf[sl]
                mask = (dvec >= lo) & (dvec < lo + BKT)
                mi = jnp.where(mask, ones_i, zeros_i)
                cs = plsc.cumsum(mi)
                idx = (off + cs - 1) & (CAP - 1)
                w = svec | ((dvec - lo) << 16)
                plsc.store_scatter(ring, [idx], w, mask=mask)
                return off + cs[15]

            off = lax.fori_loop(0, CH // 16, scan16, off)
            dr = lax.while_loop(lambda d: off - d >= RB, flush, dr)
            return (off, dr)

        off, dr = lax.fori_loop(0, nchunk, chunk,
                                (jnp.int32(0), jnp.int32(0)))

        padv = zeros_i + jnp.int32(BKT << 16)
        for k in range(RB // 16):
            idxp = (off + k * 16 + iota) & (CAP - 1)
            plsc.store_scatter(ring, [idxp], padv)
        W = (off + RB - 1) & jnp.int32(-RB)
        dr = lax.while_loop(lambda d: d < W, flush, dr)
        cbuf[...] = zeros_i + W
        pltpu.sync_copy(cbuf, cnt_out.at[pl.ds(pl.multiple_of(b * 16, 16), 16)])

    for bi in range(NBPT):
        run_bucket(wid * NBPT + bi)


def _sc_partition(src, dst):
    mesh = plsc.VectorSubcoreMesh(core_axis_name="c", subcore_axis_name="s",
                                  num_cores=NCORES, num_subcores=NSUB)
    fn = pl.kernel(
        _sc_part_body,
        out_type=[jax.ShapeDtypeStruct((NBUCK * EPB,), jnp.int32),
                  jax.ShapeDtypeStruct((NBUCK * 16,), jnp.int32)],
        mesh=mesh,
        scratch_types=[
            pltpu.VMEM((CH,), jnp.int32),
            pltpu.VMEM((CH,), jnp.int32),
            pltpu.VMEM((CAP,), jnp.int32),
            pltpu.VMEM((16,), jnp.int32),
        ],
        compiler_params=pltpu.CompilerParams(needs_layout_passes=False),
    )
    return fn(src, dst)


def _sc_stats_body(a_hbm, packed_hbm, cnt_hbm, s_out, q_out, mx_out, mn_out,
                   deg_out, pbuf, sidx0, sidx1, rows0, rows1,
                   S, Q, MX, MN, degacc, degbuf, cbuf, sem0, sem1):
    cid = lax.axis_index("c")
    sid = lax.axis_index("s")
    wid = sid * NCORES + cid
    iota = lax.iota(jnp.int32, 16)
    ones = jnp.ones((16,), jnp.float32)
    mask16 = jnp.int32(0xFFFF)

    def run_bucket(b):
        lo = b * BKT
        pbase = b * EPB
        zeros = jnp.zeros((16,), jnp.float32)
        neg = jnp.full((16,), -FINIT, jnp.float32)
        pos = jnp.full((16,), FINIT, jnp.float32)

        def initr(r, _):
            for f in range(8):
                sl = pl.ds(16 * f, 16)
                S[r, sl] = zeros
                Q[r, sl] = zeros
                MX[r, sl] = neg
                MN[r, sl] = pos
            degacc[r, :] = zeros
            return 0

        lax.fori_loop(0, BKT + 1, initr, 0)

        pltpu.sync_copy(cnt_hbm.at[pl.ds(pl.multiple_of(b * 16, 16), 16)], cbuf)
        W = cbuf[pl.ds(0, 16)][0]

        def issue(k, nb, sidx, rows, sem):
            @pl.when(k < nb)
            def _():
                kb = pl.multiple_of(k * GB, GB)
                for g in range(GB // 16):
                    wv = pbuf[pl.ds(pl.multiple_of(kb + g * 16, 16), 16)]
                    sidx[pl.ds(g * 16, 16)] = wv & mask16
                pltpu.async_copy(a_hbm.at[sidx], rows, sem)

        def waitacc(k, nb, rows, sem):
            @pl.when(k < nb)
            def _():
                pltpu.make_async_copy(a_hbm, rows, sem).wait()
                kb = pl.multiple_of(k * GB, GB)

                def group(g, _):
                    gb = pl.multiple_of(g * 16, 16)
                    wv = pbuf[pl.ds(pl.multiple_of(kb + gb, 16), 16)]
                    dlv = wv >> 16
                    for i in range(16):
                        dl = dlv[i]
                        plsc.addupdate(degacc.at[dl, :], ones)
                        for f in range(8):
                            sl = pl.ds(16 * f, 16)
                            r = rows[gb + i, sl]
                            plsc.addupdate(S.at[dl, sl], r)
                            plsc.addupdate(Q.at[dl, sl], r * r)
                            MX[dl, sl] = jnp.maximum(MX[dl, sl], r)
                            MN[dl, sl] = jnp.minimum(MN[dl, sl], r)
                    return 0

                lax.fori_loop(0, GB // 16, group, 0)

        def chunk(c, _):
            cb = c * CH2
            pltpu.sync_copy(packed_hbm.at[pl.ds(pl.multiple_of(pbase + cb, 8), CH2)], pbuf)
            nb = jnp.minimum((W - cb) // GB, CH2 // GB)
            issue(jnp.int32(0), nb, sidx0, rows0, sem0)

            def pair(p, _):
                k = p * 2
                issue(k + 1, nb, sidx1, rows1, sem1)
                waitacc(k, nb, rows0, sem0)
                issue(k + 2, nb, sidx0, rows0, sem0)
                waitacc(k + 1, nb, rows1, sem1)
                return 0

            lax.fori_loop(0, CH2 // GB // 2, pair, 0)
            return 0

        nchunk = (W + CH2 - 1) // CH2
        lax.fori_loop(0, nchunk, chunk, 0)

        lob = pl.multiple_of(lo, 8)
        pltpu.sync_copy(S.at[pl.ds(0, BKT), :], s_out.at[pl.ds(lob, BKT), :])
        pltpu.sync_copy(Q.at[pl.ds(0, BKT), :], q_out.at[pl.ds(lob, BKT), :])
        pltpu.sync_copy(MX.at[pl.ds(0, BKT), :], mx_out.at[pl.ds(lob, BKT), :])
        pltpu.sync_copy(MN.at[pl.ds(0, BKT), :], mn_out.at[pl.ds(lob, BKT), :])

        def degg(g, _):
            v = plsc.load_gather(degacc, [g * 16 + iota, jnp.zeros((16,), jnp.int32)])
            degbuf[pl.ds(pl.multiple_of(g * 16, 16), 16)] = v
            return 0

        lax.fori_loop(0, BKT // 16, degg, 0)
        pltpu.sync_copy(degbuf, deg_out.at[pl.ds(lob, BKT)])

    for bi in range(NBPT):
        run_bucket(wid * NBPT + bi)


def _sc_segment_stats(A, packed, cnts):
    mesh = plsc.VectorSubcoreMesh(core_axis_name="c", subcore_axis_name="s",
                                  num_cores=NCORES, num_subcores=NSUB)
    f32 = jnp.float32
    fn = pl.kernel(
        _sc_stats_body,
        out_type=[jax.ShapeDtypeStruct((NP, D), f32)] * 4
        + [jax.ShapeDtypeStruct((NP,), f32)],
        mesh=mesh,
        scratch_types=[
            pltpu.VMEM((CH2,), jnp.int32),
            pltpu.VMEM((GB,), jnp.int32),
            pltpu.VMEM((GB,), jnp.int32),
            pltpu.VMEM((GB, D), f32),
            pltpu.VMEM((GB, D), f32),
            pltpu.VMEM((BKT + 1, D), f32),
            pltpu.VMEM((BKT + 1, D), f32),
            pltpu.VMEM((BKT + 1, D), f32),
            pltpu.VMEM((BKT + 1, D), f32),
            pltpu.VMEM((BKT + 1, 16), f32),
            pltpu.VMEM((BKT,), f32),
            pltpu.VMEM((16,), jnp.int32),
            pltpu.SemaphoreType.DMA,
            pltpu.SemaphoreType.DMA,
        ],
        compiler_params=pltpu.CompilerParams(needs_layout_passes=False),
    )
    return fn(A, packed, cnts)


def kernel(in_feat, edge_index, Mw0, Mb0, Uw0, Ub0, MixW0, Mixb0,
           Mw1, Mb1, Uw1, Ub1, MixW1, Mixb1, Mw2, Mb2, Uw2, Ub2, MixW2, Mixb2,
           FCw, FCb):
    n = in_feat.shape[0]
    src = edge_index[0]
    dst = edge_index[1]
    h = jnp.pad(in_feat, ((0, NP - n), (0, 0)))

    packed, cnts = _sc_partition(src, dst)

    layers = [(Mw0, Mb0, Uw0, Ub0, MixW0, Mixb0),
              (Mw1, Mb1, Uw1, Ub1, MixW1, Mixb1),
              (Mw2, Mb2, Uw2, Ub2, MixW2, Mixb2)]
    degb = None
    for l, (Mw, Mb, Uw, Ub, MixW, Mixb) in enumerate(layers):
        A, B = _tc_proj(h, Mw, Mb)
        S, Q, MX, MN, deg = _sc_segment_stats(A, packed, cnts)
        if degb is None:
            degb = deg[:, None] * jnp.ones((1, D), jnp.float32)
        h = _tc_layer(h, S, Q, MX, MN, degb, B,
                      Uw, Ub, MixW, Mixb, last=(l == 2))

    FCw_pad = jnp.pad(FCw, ((0, 0), (0, D - FCw.shape[1])))
    FCb_pad = jnp.pad(FCb, (0, D - FCb.shape[0]))
    out = _tc_fc(h, FCw_pad, FCb_pad.reshape(1, D))
    return out[:n, :FCw.shape[1]]


# speedup vs baseline: 4.5973x; 1.0413x over previous
import functools
import jax
import jax.numpy as jnp
from jax import lax
from jax.experimental import pallas as pl
from jax.experimental.pallas import tpu as pltpu
from jax.experimental.pallas import tpu_sc as plsc

D = 128
DELTA = 3.5
NP = 10240
R = 1024
GRID = NP // R


def _proj_body(h_ref, mw_ref, mb_ref, a_ref, b_ref):
    h = h_ref[...]
    a_ref[...] = jnp.dot(h, mw_ref[0:D, :], preferred_element_type=jnp.float32)
    b_ref[...] = jnp.dot(h, mw_ref[D:2 * D, :], preferred_element_type=jnp.float32) + mb_ref[...]


def _tc_proj(h, Mw, Mb):
    blk = lambda i: (i, 0)
    full = lambda i: (0, 0)
    return pl.pallas_call(
        _proj_body,
        grid=(GRID,),
        in_specs=[
            pl.BlockSpec((R, D), blk),
            pl.BlockSpec((2 * D, D), full),
            pl.BlockSpec((1, D), full),
        ],
        out_specs=[pl.BlockSpec((R, D), blk), pl.BlockSpec((R, D), blk)],
        out_shape=[
            jax.ShapeDtypeStruct((NP, D), jnp.float32),
            jax.ShapeDtypeStruct((NP, D), jnp.float32),
        ],
    )(h, Mw, Mb.reshape(1, D))


def _layer_body(h_ref, s_ref, q_ref, mx_ref, mn_ref, deg_ref, b_ref,
                uw_ref, ub_ref, mixw_ref, mixb_ref, out_ref, *, last):
    h = h_ref[...]
    deg = deg_ref[...]
    B = b_ref[...]
    S = s_ref[...]
    cnt = jnp.maximum(deg, 1.0)
    pos = deg > 0.0
    mean = (S + deg * B) / cnt
    mx = jnp.where(pos, mx_ref[...] + B, 0.0)
    mn = jnp.where(pos, mn_ref[...] + B, 0.0)
    sq = (q_ref[...] + 2.0 * B * S + deg * B * B) / cnt
    var = jnp.maximum(sq - mean * mean, 0.0)
    std = jnp.sqrt(var + 1e-5)
    logd = jnp.log(deg + 1.0)
    s_amp = logd * (1.0 / DELTA)
    s_att = DELTA / jnp.maximum(logd, 1e-6)
    pieces = [h, mean, mx, mn, std,
              mean * s_amp, mx * s_amp, mn * s_amp, std * s_amp,
              mean * s_att, mx * s_att, mn * s_att, std * s_att]
    u = ub_ref[...]
    for k, p in enumerate(pieces):
        u = u + jnp.dot(p, uw_ref[k * D:(k + 1) * D, :],
                        preferred_element_type=jnp.float32)
    t = jnp.dot(u, mixw_ref[...], preferred_element_type=jnp.float32) + mixb_ref[...]
    t = jnp.where(t > 0, t, 0.01 * t)
    t = h + t
    if not last:
        t = jnp.maximum(t, 0.0)
    out_ref[...] = t


def _tc_layer(h, S, Q, MX, MN, degb, B, Uw, Ub, MixW, Mixb, last):
    blk = lambda i: (i, 0)
    full = lambda i: (0, 0)
    return pl.pallas_call(
        functools.partial(_layer_body, last=last),
        grid=(GRID,),
        in_specs=[pl.BlockSpec((R, D), blk)] * 7 + [
            pl.BlockSpec((13 * D, D), full),
            pl.BlockSpec((1, D), full),
            pl.BlockSpec((D, D), full),
            pl.BlockSpec((1, D), full),
        ],
        out_specs=pl.BlockSpec((R, D), blk),
        out_shape=jax.ShapeDtypeStruct((NP, D), jnp.float32),
    )(h, S, Q, MX, MN, degb, B, Uw, Ub.reshape(1, D), MixW, Mixb.reshape(1, D))


def _fc_body(h_ref, w_ref, b_ref, o_ref):
    o_ref[...] = jnp.dot(h_ref[...], w_ref[...],
                         preferred_element_type=jnp.float32) + b_ref[...]


def _tc_fc(h, FCw_pad, FCb_pad):
    blk = lambda i: (i, 0)
    full = lambda i: (0, 0)
    return pl.pallas_call(
        _fc_body,
        grid=(GRID,),
        in_specs=[
            pl.BlockSpec((R, D), blk),
            pl.BlockSpec((D, D), full),
            pl.BlockSpec((1, D), full),
        ],
        out_specs=pl.BlockSpec((R, D), blk),
        out_shape=jax.ShapeDtypeStruct((NP, D), jnp.float32),
    )(h, FCw_pad, FCb_pad)


NCORES = 2
NSUB = 16
NWORK = NCORES * NSUB
NBPT = 2
NBUCK = NWORK * NBPT
BKT = NP // NBUCK
CAP = 2048
RB = 128
GB = 64
CH = 1280
CH2 = 2048
EPB = 320000 + 4096
FINIT = 3.0e38


def _sc_part_body(src_hbm, dst_hbm, packed_out, cnt_out, sbuf, dbuf, stage, cbuf):
    cid = lax.axis_index("c")
    sid = lax.axis_index("s")
    wid = sid * NCORES + cid
    assert src_hbm.shape[0] % CH == 0 and CH % 16 == 0
    nchunk = src_hbm.shape[0] // CH
    zeros_i = jnp.zeros((16,), jnp.int32)

    def run_bucket(b):
        lo = b * BKT
        pbase = b * EPB

        def chunk(c, carry):
            rem, dr = carry
            cb = pl.multiple_of(c * CH, 8)
            pltpu.sync_copy(src_hbm.at[pl.ds(cb, CH)], sbuf)
            pltpu.sync_copy(dst_hbm.at[pl.ds(cb, CH)], dbuf)

            def scan16(j, off):
                sl = pl.ds(pl.multiple_of(j * 16, 16), 16)
                dvec = dbuf[sl]
                svec = sbuf[sl]
                mask = (dvec >= lo) & (dvec < lo + BKT)
                w = svec | ((dvec - lo) << 16)
                plsc.store_compressed(stage.at[pl.ds(off, 16)], w, mask=mask)
                return off + plsc.all_reduce_population_count(mask)[0]

            off = lax.fori_loop(0, CH // 16, scan16, rem)

            nfl = off // RB

            def flush(k, _):
                fb = pl.multiple_of(k * RB, RB)
                pltpu.sync_copy(stage.at[pl.ds(fb, RB)],
                                packed_out.at[pl.ds(pl.multiple_of(pbase + dr + fb, 8), RB)])
                return 0

            lax.fori_loop(0, nfl, flush, 0)
            tb = pl.multiple_of(nfl * RB, RB)
            for g in range(RB // 16):
                v = stage[pl.ds(pl.multiple_of(tb + g * 16, 16), 16)]
                stage[pl.ds(16 * g, 16)] = v
            return (off & (RB - 1), dr + nfl * RB)

        rem, dr = lax.fori_loop(0, nchunk, chunk,
                                (jnp.int32(0), jnp.int32(0)))

        padv = zeros_i + jnp.int32(BKT << 16)
        for k in range(RB // 16):
            plsc.store_compressed(stage.at[pl.ds(rem + k * 16, 16)], padv,
                                  mask=jnp.ones((16,), jnp.bool_))
        W = dr + jnp.where(rem > 0, jnp.int32(RB), jnp.int32(0))

        @pl.when(rem > 0)
        def _():
            pltpu.sync_copy(stage.at[pl.ds(0, RB)],
                            packed_out.at[pl.ds(pl.multiple_of(pbase + dr, 8), RB)])

        cbuf[...] = zeros_i + W
        pltpu.sync_copy(cbuf, cnt_out.at[pl.ds(pl.multiple_of(b * 16, 16), 16)])

    for bi in range(NBPT):
        run_bucket(wid * NBPT + bi)


def _sc_partition(src, dst):
    mesh = plsc.VectorSubcoreMesh(core_axis_name="c", subcore_axis_name="s",
                                  num_cores=NCORES, num_subcores=NSUB)
    fn = pl.kernel(
        _sc_part_body,
        out_type=[jax.ShapeDtypeStruct((NBUCK * EPB,), jnp.int32),
                  jax.ShapeDtypeStruct((NBUCK * 16,), jnp.int32)],
        mesh=mesh,
        scratch_types=[
            pltpu.VMEM((CH,), jnp.int32),
            pltpu.VMEM((CH,), jnp.int32),
            pltpu.VMEM((CH + 2 * RB,), jnp.int32),
            pltpu.VMEM((16,), jnp.int32),
        ],
        compiler_params=pltpu.CompilerParams(needs_layout_passes=False),
    )
    return fn(src, dst)


def _sc_stats_body(a_hbm, packed_hbm, cnt_hbm, s_out, q_out, mx_out, mn_out,
                   deg_out, pbuf, sidx0, sidx1, rows0, rows1,
                   S, Q, MX, MN, degacc, degbuf, cbuf, sem0, sem1):
    cid = lax.axis_index("c")
    sid = lax.axis_index("s")
    wid = sid * NCORES + cid
    iota = lax.iota(jnp.int32, 16)
    ones = jnp.ones((16,), jnp.float32)
    mask16 = jnp.int32(0xFFFF)

    def run_bucket(b):
        lo = b * BKT
        pbase = b * EPB
        zeros = jnp.zeros((16,), jnp.float32)
        neg = jnp.full((16,), -FINIT, jnp.float32)
        pos = jnp.full((16,), FINIT, jnp.float32)

        def initr(r, _):
            for f in range(8):
                sl = pl.ds(16 * f, 16)
                S[r, sl] = zeros
                Q[r, sl] = zeros
                MX[r, sl] = neg
                MN[r, sl] = pos
            degacc[r, :] = zeros
            return 0

        lax.fori_loop(0, BKT + 1, initr, 0)

        pltpu.sync_copy(cnt_hbm.at[pl.ds(pl.multiple_of(b * 16, 16), 16)], cbuf)
        W = cbuf[pl.ds(0, 16)][0]

        def issue(k, nb, sidx, rows, sem):
            @pl.when(k < nb)
            def _():
                kb = pl.multiple_of(k * GB, GB)
                for g in range(GB // 16):
                    wv = pbuf[pl.ds(pl.multiple_of(kb + g * 16, 16), 16)]
                    sidx[pl.ds(g * 16, 16)] = wv & mask16
                pltpu.async_copy(a_hbm.at[sidx], rows, sem)

        def waitacc(k, nb, rows, sem):
            @pl.when(k < nb)
            def _():
                pltpu.make_async_copy(a_hbm, rows, sem).wait()
                kb = pl.multiple_of(k * GB, GB)

                def group(g, _):
                    gb = pl.multiple_of(g * 16, 16)
                    wv = pbuf[pl.ds(pl.multiple_of(kb + gb, 16), 16)]
                    dlv = wv >> 16
                    for i in range(16):
                        dl = dlv[i]
                        plsc.addupdate(degacc.at[dl, :], ones)
                        for f in range(8):
                            sl = pl.ds(16 * f, 16)
                            r = rows[gb + i, sl]
                            plsc.addupdate(S.at[dl, sl], r)
                            plsc.addupdate(Q.at[dl, sl], r * r)
                            MX[dl, sl] = jnp.maximum(MX[dl, sl], r)
                            MN[dl, sl] = jnp.minimum(MN[dl, sl], r)
                    return 0

                lax.fori_loop(0, GB // 16, group, 0)

        def chunk(c, _):
            cb = c * CH2
            pltpu.sync_copy(packed_hbm.at[pl.ds(pl.multiple_of(pbase + cb, 8), CH2)], pbuf)
            nb = jnp.minimum((W - cb) // GB, CH2 // GB)
            issue(jnp.int32(0), nb, sidx0, rows0, sem0)

            def pair(p, _):
                k = p * 2
                issue(k + 1, nb, sidx1, rows1, sem1)
                waitacc(k, nb, rows0, sem0)
                issue(k + 2, nb, sidx0, rows0, sem0)
                waitacc(k + 1, nb, rows1, sem1)
                return 0

            lax.fori_loop(0, CH2 // GB // 2, pair, 0)
            return 0

        nchunk = (W + CH2 - 1) // CH2
        lax.fori_loop(0, nchunk, chunk, 0)

        lob = pl.multiple_of(lo, 8)
        pltpu.sync_copy(S.at[pl.ds(0, BKT), :], s_out.at[pl.ds(lob, BKT), :])
        pltpu.sync_copy(Q.at[pl.ds(0, BKT), :], q_out.at[pl.ds(lob, BKT), :])
        pltpu.sync_copy(MX.at[pl.ds(0, BKT), :], mx_out.at[pl.ds(lob, BKT), :])
        pltpu.sync_copy(MN.at[pl.ds(0, BKT), :], mn_out.at[pl.ds(lob, BKT), :])

        def degg(g, _):
            v = plsc.load_gather(degacc, [g * 16 + iota, jnp.zeros((16,), jnp.int32)])
            degbuf[pl.ds(pl.multiple_of(g * 16, 16), 16)] = v
            return 0

        lax.fori_loop(0, BKT // 16, degg, 0)
        pltpu.sync_copy(degbuf, deg_out.at[pl.ds(lob, BKT)])

    for bi in range(NBPT):
        run_bucket(wid * NBPT + bi)


def _sc_segment_stats(A, packed, cnts):
    mesh = plsc.VectorSubcoreMesh(core_axis_name="c", subcore_axis_name="s",
                                  num_cores=NCORES, num_subcores=NSUB)
    f32 = jnp.float32
    fn = pl.kernel(
        _sc_stats_body,
        out_type=[jax.ShapeDtypeStruct((NP, D), f32)] * 4
        + [jax.ShapeDtypeStruct((NP,), f32)],
        mesh=mesh,
        scratch_types=[
            pltpu.VMEM((CH2,), jnp.int32),
            pltpu.VMEM((GB,), jnp.int32),
            pltpu.VMEM((GB,), jnp.int32),
            pltpu.VMEM((GB, D), f32),
            pltpu.VMEM((GB, D), f32),
            pltpu.VMEM((BKT + 1, D), f32),
            pltpu.VMEM((BKT + 1, D), f32),
            pltpu.VMEM((BKT + 1, D), f32),
            pltpu.VMEM((BKT + 1, D), f32),
            pltpu.VMEM((BKT + 1, 16), f32),
            pltpu.VMEM((BKT,), f32),
            pltpu.VMEM((16,), jnp.int32),
            pltpu.SemaphoreType.DMA,
            pltpu.SemaphoreType.DMA,
        ],
        compiler_params=pltpu.CompilerParams(needs_layout_passes=False),
    )
    return fn(A, packed, cnts)


def kernel(in_feat, edge_index, Mw0, Mb0, Uw0, Ub0, MixW0, Mixb0,
           Mw1, Mb1, Uw1, Ub1, MixW1, Mixb1, Mw2, Mb2, Uw2, Ub2, MixW2, Mixb2,
           FCw, FCb):
    n = in_feat.shape[0]
    src = edge_index[0]
    dst = edge_index[1]
    h = jnp.pad(in_feat, ((0, NP - n), (0, 0)))

    packed, cnts = _sc_partition(src, dst)

    layers = [(Mw0, Mb0, Uw0, Ub0, MixW0, Mixb0),
              (Mw1, Mb1, Uw1, Ub1, MixW1, Mixb1),
              (Mw2, Mb2, Uw2, Ub2, MixW2, Mixb2)]
    degb = None
    for l, (Mw, Mb, Uw, Ub, MixW, Mixb) in enumerate(layers):
        A, B = _tc_proj(h, Mw, Mb)
        S, Q, MX, MN, deg = _sc_segment_stats(A, packed, cnts)
        if degb is None:
            degb = deg[:, None] * jnp.ones((1, D), jnp.float32)
        h = _tc_layer(h, S, Q, MX, MN, degb, B,
                      Uw, Ub, MixW, Mixb, last=(l == 2))

    FCw_pad = jnp.pad(FCw, ((0, 0), (0, D - FCw.shape[1])))
    FCb_pad = jnp.pad(FCb, (0, D - FCb.shape[0]))
    out = _tc_fc(h, FCw_pad, FCb_pad.reshape(1, D))
    return out[:n, :FCw.shape[1]]


# speedup vs baseline: 5.5067x; 1.1978x over previous
import functools
import jax
import jax.numpy as jnp
from jax import lax
from jax.experimental import pallas as pl
from jax.experimental.pallas import tpu as pltpu
from jax.experimental.pallas import tpu_sc as plsc

D = 128
DELTA = 3.5
NP = 10240
R = 1024
GRID = NP // R


def _proj_body(h_ref, mw_ref, mb_ref, a_ref, b_ref):
    h = h_ref[...]
    a_ref[...] = jnp.dot(h, mw_ref[0:D, :], preferred_element_type=jnp.float32)
    b_ref[...] = jnp.dot(h, mw_ref[D:2 * D, :], preferred_element_type=jnp.float32) + mb_ref[...]


def _tc_proj(h, Mw, Mb):
    blk = lambda i: (i, 0)
    full = lambda i: (0, 0)
    return pl.pallas_call(
        _proj_body,
        grid=(GRID,),
        in_specs=[
            pl.BlockSpec((R, D), blk),
            pl.BlockSpec((2 * D, D), full),
            pl.BlockSpec((1, D), full),
        ],
        out_specs=[pl.BlockSpec((R, D), blk), pl.BlockSpec((R, D), blk)],
        out_shape=[
            jax.ShapeDtypeStruct((NP, D), jnp.float32),
            jax.ShapeDtypeStruct((NP, D), jnp.float32),
        ],
    )(h, Mw, Mb.reshape(1, D))


def _layer_body(h_ref, s_ref, q_ref, mx_ref, mn_ref, deg_ref, b_ref,
                uw_ref, ub_ref, mixw_ref, mixb_ref, out_ref, *, last):
    h = h_ref[...]
    deg = deg_ref[...]
    B = b_ref[...]
    S = s_ref[...]
    cnt = jnp.maximum(deg, 1.0)
    pos = deg > 0.0
    mean = (S + deg * B) / cnt
    mx = jnp.where(pos, mx_ref[...] + B, 0.0)
    mn = jnp.where(pos, mn_ref[...] + B, 0.0)
    sq = (q_ref[...] + 2.0 * B * S + deg * B * B) / cnt
    var = jnp.maximum(sq - mean * mean, 0.0)
    std = jnp.sqrt(var + 1e-5)
    logd = jnp.log(deg + 1.0)
    s_amp = logd * (1.0 / DELTA)
    s_att = DELTA / jnp.maximum(logd, 1e-6)
    pieces = [h, mean, mx, mn, std,
              mean * s_amp, mx * s_amp, mn * s_amp, std * s_amp,
              mean * s_att, mx * s_att, mn * s_att, std * s_att]
    u = ub_ref[...]
    for k, p in enumerate(pieces):
        u = u + jnp.dot(p, uw_ref[k * D:(k + 1) * D, :],
                        preferred_element_type=jnp.float32)
    t = jnp.dot(u, mixw_ref[...], preferred_element_type=jnp.float32) + mixb_ref[...]
    t = jnp.where(t > 0, t, 0.01 * t)
    t = h + t
    if not last:
        t = jnp.maximum(t, 0.0)
    out_ref[...] = t


def _tc_layer(h, S, Q, MX, MN, degb, B, Uw, Ub, MixW, Mixb, last):
    blk = lambda i: (i, 0)
    full = lambda i: (0, 0)
    return pl.pallas_call(
        functools.partial(_layer_body, last=last),
        grid=(GRID,),
        in_specs=[pl.BlockSpec((R, D), blk)] * 7 + [
            pl.BlockSpec((13 * D, D), full),
            pl.BlockSpec((1, D), full),
            pl.BlockSpec((D, D), full),
            pl.BlockSpec((1, D), full),
        ],
        out_specs=pl.BlockSpec((R, D), blk),
        out_shape=jax.ShapeDtypeStruct((NP, D), jnp.float32),
    )(h, S, Q, MX, MN, degb, B, Uw, Ub.reshape(1, D), MixW, Mixb.reshape(1, D))


def _fc_body(h_ref, w_ref, b_ref, o_ref):
    o_ref[...] = jnp.dot(h_ref[...], w_ref[...],
                         preferred_element_type=jnp.float32) + b_ref[...]


def _tc_fc(h, FCw_pad, FCb_pad):
    blk = lambda i: (i, 0)
    full = lambda i: (0, 0)
    return pl.pallas_call(
        _fc_body,
        grid=(GRID,),
        in_specs=[
            pl.BlockSpec((R, D), blk),
            pl.BlockSpec((D, D), full),
            pl.BlockSpec((1, D), full),
        ],
        out_specs=pl.BlockSpec((R, D), blk),
        out_shape=jax.ShapeDtypeStruct((NP, D), jnp.float32),
    )(h, FCw_pad, FCb_pad)


NCORES = 2
NSUB = 16
NWORK = NCORES * NSUB
NBPT = 2
NBUCK = NWORK * NBPT
BKT = NP // NBUCK
CAP = 2048
RB = 128
GB = 64
CH = 1280
CH2 = 2048
EPB = 320000 + 4096
FINIT = 3.0e38


def _sc_part_body(src_hbm, dst_hbm, packed_out, cnt_out, sbuf, dbuf,
                  stage0, stage1, cbuf):
    cid = lax.axis_index("c")
    sid = lax.axis_index("s")
    wid = sid * NCORES + cid
    assert src_hbm.shape[0] % CH == 0 and CH % 16 == 0
    nchunk = src_hbm.shape[0] // CH
    zeros_i = jnp.zeros((16,), jnp.int32)
    full_m = jnp.ones((16,), jnp.bool_)

    b0 = wid * NBPT
    lo0 = b0 * BKT
    pb0 = b0 * EPB
    pb1 = (b0 + 1) * EPB

    def drain(stage, off, dr, pbase):
        nfl = off // RB

        def flush(k, _):
            fb = pl.multiple_of(k * RB, RB)
            pltpu.sync_copy(stage.at[pl.ds(fb, RB)],
                            packed_out.at[pl.ds(pl.multiple_of(pbase + dr + fb, 8), RB)])
            return 0

        lax.fori_loop(0, nfl, flush, 0)
        tb = pl.multiple_of(nfl * RB, RB)
        for g in range(RB // 16):
            v = stage[pl.ds(pl.multiple_of(tb + g * 16, 16), 16)]
            stage[pl.ds(16 * g, 16)] = v
        return off & (RB - 1), dr + nfl * RB

    def chunk(c, carry):
        rem0, dr0, rem1, dr1 = carry
        cb = pl.multiple_of(c * CH, 8)
        pltpu.sync_copy(src_hbm.at[pl.ds(cb, CH)], sbuf)
        pltpu.sync_copy(dst_hbm.at[pl.ds(cb, CH)], dbuf)

        def scan16(j, offs):
            off0, off1 = offs
            sl = pl.ds(pl.multiple_of(j * 16, 16), 16)
            dvec = dbuf[sl]
            svec = sbuf[sl]
            dl = dvec - lo0
            m0 = (dl >= 0) & (dl < BKT)
            m1 = (dl >= BKT) & (dl < 2 * BKT)
            w0 = svec | (dl << 16)
            w1 = w0 - jnp.int32(BKT << 16)
            plsc.store_compressed(stage0.at[pl.ds(off0, 16)], w0, mask=m0)
            plsc.store_compressed(stage1.at[pl.ds(off1, 16)], w1, mask=m1)
            return (off0 + plsc.all_reduce_population_count(m0)[0],
                    off1 + plsc.all_reduce_population_count(m1)[0])

        off0, off1 = lax.fori_loop(0, CH // 16, scan16, (rem0, rem1))
        rem0, dr0 = drain(stage0, off0, dr0, pb0)
        rem1, dr1 = drain(stage1, off1, dr1, pb1)
        return (rem0, dr0, rem1, dr1)

    z = jnp.int32(0)
    rem0, dr0, rem1, dr1 = lax.fori_loop(0, nchunk, chunk, (z, z, z, z))

    def finish(stage, rem, dr, pbase, b):
        padv = zeros_i + jnp.int32(BKT << 16)
        for k in range(RB // 16):
            plsc.store_compressed(stage.at[pl.ds(rem + k * 16, 16)], padv,
                                  mask=full_m)
        W = dr + jnp.where(rem > 0, jnp.int32(RB), jnp.int32(0))

        @pl.when(rem > 0)
        def _():
            pltpu.sync_copy(stage.at[pl.ds(0, RB)],
                            packed_out.at[pl.ds(pl.multiple_of(pbase + dr, 8), RB)])

        cbuf[...] = zeros_i + W
        pltpu.sync_copy(cbuf, cnt_out.at[pl.ds(pl.multiple_of(b * 16, 16), 16)])

    finish(stage0, rem0, dr0, pb0, b0)
    finish(stage1, rem1, dr1, pb1, b0 + 1)


def _sc_partition(src, dst):
    mesh = plsc.VectorSubcoreMesh(core_axis_name="c", subcore_axis_name="s",
                                  num_cores=NCORES, num_subcores=NSUB)
    fn = pl.kernel(
        _sc_part_body,
        out_type=[jax.ShapeDtypeStruct((NBUCK * EPB,), jnp.int32),
                  jax.ShapeDtypeStruct((NBUCK * 16,), jnp.int32)],
        mesh=mesh,
        scratch_types=[
            pltpu.VMEM((CH,), jnp.int32),
            pltpu.VMEM((CH,), jnp.int32),
            pltpu.VMEM((CH + 2 * RB,), jnp.int32),
            pltpu.VMEM((CH + 2 * RB,), jnp.int32),
            pltpu.VMEM((16,), jnp.int32),
        ],
        compiler_params=pltpu.CompilerParams(needs_layout_passes=False),
    )
    return fn(src, dst)


def _sc_stats_body(a_hbm, packed_hbm, cnt_hbm, s_out, q_out, mx_out, mn_out,
                   deg_out, pbuf, sidx0, sidx1, rows0, rows1,
                   S, Q, MX, MN, degacc, degbuf, cbuf, sem0, sem1):
    cid = lax.axis_index("c")
    sid = lax.axis_index("s")
    wid = sid * NCORES + cid
    iota = lax.iota(jnp.int32, 16)
    ones = jnp.ones((16,), jnp.float32)
    mask16 = jnp.int32(0xFFFF)

    def run_bucket(b):
        lo = b * BKT
        pbase = b * EPB
        zeros = jnp.zeros((16,), jnp.float32)
        neg = jnp.full((16,), -FINIT, jnp.float32)
        pos = jnp.full((16,), FINIT, jnp.float32)

        def initr(r, _):
            for f in range(8):
                sl = pl.ds(16 * f, 16)
                S[r, sl] = zeros
                Q[r, sl] = zeros
                MX[r, sl] = neg
                MN[r, sl] = pos
            degacc[r, :] = zeros
            return 0

        lax.fori_loop(0, BKT + 1, initr, 0)

        pltpu.sync_copy(cnt_hbm.at[pl.ds(pl.multiple_of(b * 16, 16), 16)], cbuf)
        W = cbuf[pl.ds(0, 16)][0]

        def issue(k, nb, sidx, rows, sem):
            @pl.when(k < nb)
            def _():
                kb = pl.multiple_of(k * GB, GB)
                for g in range(GB // 16):
                    wv = pbuf[pl.ds(pl.multiple_of(kb + g * 16, 16), 16)]
                    sidx[pl.ds(g * 16, 16)] = wv & mask16
                pltpu.async_copy(a_hbm.at[sidx], rows, sem)

        def waitacc(k, nb, rows, sem):
            @pl.when(k < nb)
            def _():
                pltpu.make_async_copy(a_hbm, rows, sem).wait()
                kb = pl.multiple_of(k * GB, GB)

                def group(g, _):
                    gb = pl.multiple_of(g * 16, 16)
                    wv = pbuf[pl.ds(pl.multiple_of(kb + gb, 16), 16)]
                    dlv = wv >> 16
                    for i in range(16):
                        dl = dlv[i]
                        plsc.addupdate(degacc.at[dl, :], ones)
                        for f in range(8):
                            sl = pl.ds(16 * f, 16)
                            r = rows[gb + i, sl]
                            plsc.addupdate(S.at[dl, sl], r)
                            plsc.addupdate(Q.at[dl, sl], r * r)
                            MX[dl, sl] = jnp.maximum(MX[dl, sl], r)
                            MN[dl, sl] = jnp.minimum(MN[dl, sl], r)
                    return 0

                lax.fori_loop(0, GB // 16, group, 0)

        def chunk(c, _):
            cb = c * CH2
            pltpu.sync_copy(packed_hbm.at[pl.ds(pl.multiple_of(pbase + cb, 8), CH2)], pbuf)
            nb = jnp.minimum((W - cb) // GB, CH2 // GB)
            issue(jnp.int32(0), nb, sidx0, rows0, sem0)

            def pair(p, _):
                k = p * 2
                issue(k + 1, nb, sidx1, rows1, sem1)
                waitacc(k, nb, rows0, sem0)
                issue(k + 2, nb, sidx0, rows0, sem0)
                waitacc(k + 1, nb, rows1, sem1)
                return 0

            lax.fori_loop(0, CH2 // GB // 2, pair, 0)
            return 0

        nchunk = (W + CH2 - 1) // CH2
        lax.fori_loop(0, nchunk, chunk, 0)

        lob = pl.multiple_of(lo, 8)
        pltpu.sync_copy(S.at[pl.ds(0, BKT), :], s_out.at[pl.ds(lob, BKT), :])
        pltpu.sync_copy(Q.at[pl.ds(0, BKT), :], q_out.at[pl.ds(lob, BKT), :])
        pltpu.sync_copy(MX.at[pl.ds(0, BKT), :], mx_out.at[pl.ds(lob, BKT), :])
        pltpu.sync_copy(MN.at[pl.ds(0, BKT), :], mn_out.at[pl.ds(lob, BKT), :])

        def degg(g, _):
            v = plsc.load_gather(degacc, [g * 16 + iota, jnp.zeros((16,), jnp.int32)])
            degbuf[pl.ds(pl.multiple_of(g * 16, 16), 16)] = v
            return 0

        lax.fori_loop(0, BKT // 16, degg, 0)
        pltpu.sync_copy(degbuf, deg_out.at[pl.ds(lob, BKT)])

    for bi in range(NBPT):
        run_bucket(wid * NBPT + bi)


def _sc_segment_stats(A, packed, cnts):
    mesh = plsc.VectorSubcoreMesh(core_axis_name="c", subcore_axis_name="s",
                                  num_cores=NCORES, num_subcores=NSUB)
    f32 = jnp.float32
    fn = pl.kernel(
        _sc_stats_body,
        out_type=[jax.ShapeDtypeStruct((NP, D), f32)] * 4
        + [jax.ShapeDtypeStruct((NP,), f32)],
        mesh=mesh,
        scratch_types=[
            pltpu.VMEM((CH2,), jnp.int32),
            pltpu.VMEM((GB,), jnp.int32),
            pltpu.VMEM((GB,), jnp.int32),
            pltpu.VMEM((GB, D), f32),
            pltpu.VMEM((GB, D), f32),
            pltpu.VMEM((BKT + 1, D), f32),
            pltpu.VMEM((BKT + 1, D), f32),
            pltpu.VMEM((BKT + 1, D), f32),
            pltpu.VMEM((BKT + 1, D), f32),
            pltpu.VMEM((BKT + 1, 16), f32),
            pltpu.VMEM((BKT,), f32),
            pltpu.VMEM((16,), jnp.int32),
            pltpu.SemaphoreType.DMA,
            pltpu.SemaphoreType.DMA,
        ],
        compiler_params=pltpu.CompilerParams(needs_layout_passes=False),
    )
    return fn(A, packed, cnts)


def kernel(in_feat, edge_index, Mw0, Mb0, Uw0, Ub0, MixW0, Mixb0,
           Mw1, Mb1, Uw1, Ub1, MixW1, Mixb1, Mw2, Mb2, Uw2, Ub2, MixW2, Mixb2,
           FCw, FCb):
    n = in_feat.shape[0]
    src = edge_index[0]
    dst = edge_index[1]
    h = jnp.pad(in_feat, ((0, NP - n), (0, 0)))

    packed, cnts = _sc_partition(src, dst)

    layers = [(Mw0, Mb0, Uw0, Ub0, MixW0, Mixb0),
              (Mw1, Mb1, Uw1, Ub1, MixW1, Mixb1),
              (Mw2, Mb2, Uw2, Ub2, MixW2, Mixb2)]
    degb = None
    for l, (Mw, Mb, Uw, Ub, MixW, Mixb) in enumerate(layers):
        A, B = _tc_proj(h, Mw, Mb)
        S, Q, MX, MN, deg = _sc_segment_stats(A, packed, cnts)
        if degb is None:
            degb = deg[:, None] * jnp.ones((1, D), jnp.float32)
        h = _tc_layer(h, S, Q, MX, MN, degb, B,
                      Uw, Ub, MixW, Mixb, last=(l == 2))

    FCw_pad = jnp.pad(FCw, ((0, 0), (0, D - FCw.shape[1])))
    FCb_pad = jnp.pad(FCb, (0, D - FCb.shape[0]))
    out = _tc_fc(h, FCw_pad, FCb_pad.reshape(1, D))
    return out[:n, :FCw.shape[1]]


# speedup vs baseline: 6.2406x; 1.1333x over previous
import functools
import jax
import jax.numpy as jnp
from jax import lax
from jax.experimental import pallas as pl
from jax.experimental.pallas import tpu as pltpu
from jax.experimental.pallas import tpu_sc as plsc

D = 128
DELTA = 3.5
NP = 10240
R = 1024
GRID = NP // R


def _proj_body(h_ref, mw_ref, mb_ref, a_ref, b_ref):
    h = h_ref[...]
    a_ref[...] = jnp.dot(h, mw_ref[0:D, :], preferred_element_type=jnp.float32)
    b_ref[...] = jnp.dot(h, mw_ref[D:2 * D, :], preferred_element_type=jnp.float32) + mb_ref[...]


def _tc_proj(h, Mw, Mb):
    blk = lambda i: (i, 0)
    full = lambda i: (0, 0)
    return pl.pallas_call(
        _proj_body,
        grid=(GRID,),
        in_specs=[
            pl.BlockSpec((R, D), blk),
            pl.BlockSpec((2 * D, D), full),
            pl.BlockSpec((1, D), full),
        ],
        out_specs=[pl.BlockSpec((R, D), blk), pl.BlockSpec((R, D), blk)],
        out_shape=[
            jax.ShapeDtypeStruct((NP, D), jnp.float32),
            jax.ShapeDtypeStruct((NP, D), jnp.float32),
        ],
    )(h, Mw, Mb.reshape(1, D))


def _layer_body(h_ref, s_ref, q_ref, mx_ref, mn_ref, deg_ref, b_ref,
                uw_ref, ub_ref, mixw_ref, mixb_ref, out_ref, *, last):
    h = h_ref[...]
    deg = deg_ref[...]
    B = b_ref[...]
    S = s_ref[...]
    cnt = jnp.maximum(deg, 1.0)
    pos = deg > 0.0
    mean = (S + deg * B) / cnt
    mx = jnp.where(pos, mx_ref[...] + B, 0.0)
    mn = jnp.where(pos, mn_ref[...] + B, 0.0)
    sq = (q_ref[...] + 2.0 * B * S + deg * B * B) / cnt
    var = jnp.maximum(sq - mean * mean, 0.0)
    std = jnp.sqrt(var + 1e-5)
    logd = jnp.log(deg + 1.0)
    s_amp = logd * (1.0 / DELTA)
    s_att = DELTA / jnp.maximum(logd, 1e-6)
    pieces = [h, mean, mx, mn, std,
              mean * s_amp, mx * s_amp, mn * s_amp, std * s_amp,
              mean * s_att, mx * s_att, mn * s_att, std * s_att]
    u = ub_ref[...]
    for k, p in enumerate(pieces):
        u = u + jnp.dot(p, uw_ref[k * D:(k + 1) * D, :],
                        preferred_element_type=jnp.float32)
    t = jnp.dot(u, mixw_ref[...], preferred_element_type=jnp.float32) + mixb_ref[...]
    t = jnp.where(t > 0, t, 0.01 * t)
    t = h + t
    if not last:
        t = jnp.maximum(t, 0.0)
    out_ref[...] = t


def _tc_layer(h, S, Q, MX, MN, degb, B, Uw, Ub, MixW, Mixb, last):
    blk = lambda i: (i, 0)
    full = lambda i: (0, 0)
    return pl.pallas_call(
        functools.partial(_layer_body, last=last),
        grid=(GRID,),
        in_specs=[pl.BlockSpec((R, D), blk)] * 7 + [
            pl.BlockSpec((13 * D, D), full),
            pl.BlockSpec((1, D), full),
            pl.BlockSpec((D, D), full),
            pl.BlockSpec((1, D), full),
        ],
        out_specs=pl.BlockSpec((R, D), blk),
        out_shape=jax.ShapeDtypeStruct((NP, D), jnp.float32),
    )(h, S, Q, MX, MN, degb, B, Uw, Ub.reshape(1, D), MixW, Mixb.reshape(1, D))


def _fc_body(h_ref, w_ref, b_ref, o_ref):
    o_ref[...] = jnp.dot(h_ref[...], w_ref[...],
                         preferred_element_type=jnp.float32) + b_ref[...]


def _tc_fc(h, FCw_pad, FCb_pad):
    blk = lambda i: (i, 0)
    full = lambda i: (0, 0)
    return pl.pallas_call(
        _fc_body,
        grid=(GRID,),
        in_specs=[
            pl.BlockSpec((R, D), blk),
            pl.BlockSpec((D, D), full),
            pl.BlockSpec((1, D), full),
        ],
        out_specs=pl.BlockSpec((R, D), blk),
        out_shape=jax.ShapeDtypeStruct((NP, D), jnp.float32),
    )(h, FCw_pad, FCb_pad)


NCORES = 2
NSUB = 16
NWORK = NCORES * NSUB
NBPT = 2
NBUCK = NWORK * NBPT
BKT = NP // NBUCK
CAP = 2048
RB = 128
GB = 64
CH = 1280
CH2 = 2048
EPB = 320000 + 4096
FINIT = 3.0e38


def _sc_part_body(src_hbm, dst_hbm, packed_out, cnt_out, sbuf0, dbuf0,
                  sbuf1, dbuf1, stage0, stage1, cbuf,
                  semS0, semD0, semS1, semD1):
    cid = lax.axis_index("c")
    sid = lax.axis_index("s")
    wid = sid * NCORES + cid
    assert src_hbm.shape[0] % CH == 0 and CH % 16 == 0
    nchunk = src_hbm.shape[0] // CH
    assert nchunk % 2 == 0
    zeros_i = jnp.zeros((16,), jnp.int32)
    full_m = jnp.ones((16,), jnp.bool_)

    b0 = wid * NBPT
    lo0 = b0 * BKT
    pb0 = b0 * EPB
    pb1 = (b0 + 1) * EPB

    def drain(stage, off, dr, pbase):
        nfl = off // RB

        def flush(k, _):
            fb = pl.multiple_of(k * RB, RB)
            pltpu.sync_copy(stage.at[pl.ds(fb, RB)],
                            packed_out.at[pl.ds(pl.multiple_of(pbase + dr + fb, 8), RB)])
            return 0

        lax.fori_loop(0, nfl, flush, 0)
        tb = pl.multiple_of(nfl * RB, RB)
        for g in range(RB // 16):
            v = stage[pl.ds(pl.multiple_of(tb + g * 16, 16), 16)]
            stage[pl.ds(16 * g, 16)] = v
        return off & (RB - 1), dr + nfl * RB

    def load(c, sb, db, ss, sd):
        @pl.when(c < nchunk)
        def _():
            cb = pl.multiple_of(c * CH, 8)
            pltpu.async_copy(src_hbm.at[pl.ds(cb, CH)], sb, ss)
            pltpu.async_copy(dst_hbm.at[pl.ds(cb, CH)], db, sd)

    def scan_chunk(sbuf, dbuf, carry):
        rem0, dr0, rem1, dr1 = carry

        def scan16(j, offs):
            off0, off1 = offs
            sl = pl.ds(pl.multiple_of(j * 16, 16), 16)
            dvec = dbuf[sl]
            svec = sbuf[sl]
            dl = dvec - lo0
            m0 = (dl >= 0) & (dl < BKT)
            m1 = (dl >= BKT) & (dl < 2 * BKT)
            w0 = svec | (dl << 16)
            w1 = w0 - jnp.int32(BKT << 16)
            plsc.store_compressed(stage0.at[pl.ds(off0, 16)], w0, mask=m0)
            plsc.store_compressed(stage1.at[pl.ds(off1, 16)], w1, mask=m1)
            return (off0 + plsc.all_reduce_population_count(m0)[0],
                    off1 + plsc.all_reduce_population_count(m1)[0])

        off0, off1 = lax.fori_loop(0, CH // 16, scan16, (rem0, rem1))
        rem0, dr0 = drain(stage0, off0, dr0, pb0)
        rem1, dr1 = drain(stage1, off1, dr1, pb1)
        return (rem0, dr0, rem1, dr1)

    load(jnp.int32(0), sbuf0, dbuf0, semS0, semD0)

    def pairloop(p, carry):
        c = p * 2
        load(c + 1, sbuf1, dbuf1, semS1, semD1)
        pltpu.make_async_copy(src_hbm, sbuf0, semS0).wait()
        pltpu.make_async_copy(dst_hbm, dbuf0, semD0).wait()
        carry = scan_chunk(sbuf0, dbuf0, carry)
        load(c + 2, sbuf0, dbuf0, semS0, semD0)
        pltpu.make_async_copy(src_hbm, sbuf1, semS1).wait()
        pltpu.make_async_copy(dst_hbm, dbuf1, semD1).wait()
        carry = scan_chunk(sbuf1, dbuf1, carry)
        return carry

    z = jnp.int32(0)
    rem0, dr0, rem1, dr1 = lax.fori_loop(0, nchunk // 2, pairloop,
                                         (z, z, z, z))

    def finish(stage, rem, dr, pbase, b):
        padv = zeros_i + jnp.int32(BKT << 16)
        for k in range(RB // 16):
            plsc.store_compressed(stage.at[pl.ds(rem + k * 16, 16)], padv,
                                  mask=full_m)
        W = dr + jnp.where(rem > 0, jnp.int32(RB), jnp.int32(0))

        @pl.when(rem > 0)
        def _():
            pltpu.sync_copy(stage.at[pl.ds(0, RB)],
                            packed_out.at[pl.ds(pl.multiple_of(pbase + dr, 8), RB)])

        cbuf[...] = zeros_i + W
        pltpu.sync_copy(cbuf, cnt_out.at[pl.ds(pl.multiple_of(b * 16, 16), 16)])

    finish(stage0, rem0, dr0, pb0, b0)
    finish(stage1, rem1, dr1, pb1, b0 + 1)


def _sc_partition(src, dst):
    mesh = plsc.VectorSubcoreMesh(core_axis_name="c", subcore_axis_name="s",
                                  num_cores=NCORES, num_subcores=NSUB)
    fn = pl.kernel(
        _sc_part_body,
        out_type=[jax.ShapeDtypeStruct((NBUCK * EPB,), jnp.int32),
                  jax.ShapeDtypeStruct((NBUCK * 16,), jnp.int32)],
        mesh=mesh,
        scratch_types=[
            pltpu.VMEM((CH,), jnp.int32),
            pltpu.VMEM((CH,), jnp.int32),
            pltpu.VMEM((CH,), jnp.int32),
            pltpu.VMEM((CH,), jnp.int32),
            pltpu.VMEM((CH + 2 * RB,), jnp.int32),
            pltpu.VMEM((CH + 2 * RB,), jnp.int32),
            pltpu.VMEM((16,), jnp.int32),
            pltpu.SemaphoreType.DMA,
            pltpu.SemaphoreType.DMA,
            pltpu.SemaphoreType.DMA,
            pltpu.SemaphoreType.DMA,
        ],
        compiler_params=pltpu.CompilerParams(needs_layout_passes=False),
    )
    return fn(src, dst)


def _sc_stats_body(a_hbm, packed_hbm, cnt_hbm, s_out, q_out, mx_out, mn_out,
                   deg_out, pbuf, sidx0, sidx1, rows0, rows1,
                   S, Q, MX, MN, degacc, degbuf, cbuf, sem0, sem1):
    cid = lax.axis_index("c")
    sid = lax.axis_index("s")
    wid = sid * NCORES + cid
    iota = lax.iota(jnp.int32, 16)
    ones = jnp.ones((16,), jnp.float32)
    mask16 = jnp.int32(0xFFFF)

    def run_bucket(b):
        lo = b * BKT
        pbase = b * EPB
        zeros = jnp.zeros((16,), jnp.float32)
        neg = jnp.full((16,), -FINIT, jnp.float32)
        pos = jnp.full((16,), FINIT, jnp.float32)

        def initr(r, _):
            for f in range(8):
                sl = pl.ds(16 * f, 16)
                S[r, sl] = zeros
                Q[r, sl] = zeros
                MX[r, sl] = neg
                MN[r, sl] = pos
            degacc[r, :] = zeros
            return 0

        lax.fori_loop(0, BKT + 1, initr, 0)

        pltpu.sync_copy(cnt_hbm.at[pl.ds(pl.multiple_of(b * 16, 16), 16)], cbuf)
        W = cbuf[pl.ds(0, 16)][0]

        def issue(k, nb, sidx, rows, sem):
            @pl.when(k < nb)
            def _():
                kb = pl.multiple_of(k * GB, GB)
                for g in range(GB // 16):
                    wv = pbuf[pl.ds(pl.multiple_of(kb + g * 16, 16), 16)]
                    sidx[pl.ds(g * 16, 16)] = wv & mask16
                pltpu.async_copy(a_hbm.at[sidx], rows, sem)

        def waitacc(k, nb, rows, sem):
            @pl.when(k < nb)
            def _():
                pltpu.make_async_copy(a_hbm, rows, sem).wait()
                kb = pl.multiple_of(k * GB, GB)

                def group(g, _):
                    gb = pl.multiple_of(g * 16, 16)
                    wv = pbuf[pl.ds(pl.multiple_of(kb + gb, 16), 16)]
                    dlv = wv >> 16
                    for i in range(16):
                        dl = dlv[i]
                        plsc.addupdate(degacc.at[dl, :], ones)
                        for f in range(8):
                            sl = pl.ds(16 * f, 16)
                            r = rows[gb + i, sl]
                            plsc.addupdate(S.at[dl, sl], r)
                            plsc.addupdate(Q.at[dl, sl], r * r)
                            MX[dl, sl] = jnp.maximum(MX[dl, sl], r)
                            MN[dl, sl] = jnp.minimum(MN[dl, sl], r)
                    return 0

                lax.fori_loop(0, GB // 16, group, 0)

        def chunk(c, _):
            cb = c * CH2
            pltpu.sync_copy(packed_hbm.at[pl.ds(pl.multiple_of(pbase + cb, 8), CH2)], pbuf)
            nb = jnp.minimum((W - cb) // GB, CH2 // GB)
            issue(jnp.int32(0), nb, sidx0, rows0, sem0)

            def pair(p, _):
                k = p * 2
                issue(k + 1, nb, sidx1, rows1, sem1)
                waitacc(k, nb, rows0, sem0)
                issue(k + 2, nb, sidx0, rows0, sem0)
                waitacc(k + 1, nb, rows1, sem1)
                return 0

            lax.fori_loop(0, CH2 // GB // 2, pair, 0)
            return 0

        nchunk = (W + CH2 - 1) // CH2
        lax.fori_loop(0, nchunk, chunk, 0)

        lob = pl.multiple_of(lo, 8)
        pltpu.sync_copy(S.at[pl.ds(0, BKT), :], s_out.at[pl.ds(lob, BKT), :])
        pltpu.sync_copy(Q.at[pl.ds(0, BKT), :], q_out.at[pl.ds(lob, BKT), :])
        pltpu.sync_copy(MX.at[pl.ds(0, BKT), :], mx_out.at[pl.ds(lob, BKT), :])
        pltpu.sync_copy(MN.at[pl.ds(0, BKT), :], mn_out.at[pl.ds(lob, BKT), :])

        def degg(g, _):
            v = plsc.load_gather(degacc, [g * 16 + iota, jnp.zeros((16,), jnp.int32)])
            degbuf[pl.ds(pl.multiple_of(g * 16, 16), 16)] = v
            return 0

        lax.fori_loop(0, BKT // 16, degg, 0)
        pltpu.sync_copy(degbuf, deg_out.at[pl.ds(lob, BKT)])

    for bi in range(NBPT):
        run_bucket(wid * NBPT + bi)


def _sc_segment_stats(A, packed, cnts):
    mesh = plsc.VectorSubcoreMesh(core_axis_name="c", subcore_axis_name="s",
                                  num_cores=NCORES, num_subcores=NSUB)
    f32 = jnp.float32
    fn = pl.kernel(
        _sc_stats_body,
        out_type=[jax.ShapeDtypeStruct((NP, D), f32)] * 4
        + [jax.ShapeDtypeStruct((NP,), f32)],
        mesh=mesh,
        scratch_types=[
            pltpu.VMEM((CH2,), jnp.int32),
            pltpu.VMEM((GB,), jnp.int32),
            pltpu.VMEM((GB,), jnp.int32),
            pltpu.VMEM((GB, D), f32),
            pltpu.VMEM((GB, D), f32),
            pltpu.VMEM((BKT + 1, D), f32),
            pltpu.VMEM((BKT + 1, D), f32),
            pltpu.VMEM((BKT + 1, D), f32),
            pltpu.VMEM((BKT + 1, D), f32),
            pltpu.VMEM((BKT + 1, 16), f32),
            pltpu.VMEM((BKT,), f32),
            pltpu.VMEM((16,), jnp.int32),
            pltpu.SemaphoreType.DMA,
            pltpu.SemaphoreType.DMA,
        ],
        compiler_params=pltpu.CompilerParams(needs_layout_passes=False),
    )
    return fn(A, packed, cnts)


def kernel(in_feat, edge_index, Mw0, Mb0, Uw0, Ub0, MixW0, Mixb0,
           Mw1, Mb1, Uw1, Ub1, MixW1, Mixb1, Mw2, Mb2, Uw2, Ub2, MixW2, Mixb2,
           FCw, FCb):
    n = in_feat.shape[0]
    src = edge_index[0]
    dst = edge_index[1]
    h = jnp.pad(in_feat, ((0, NP - n), (0, 0)))

    packed, cnts = _sc_partition(src, dst)

    layers = [(Mw0, Mb0, Uw0, Ub0, MixW0, Mixb0),
              (Mw1, Mb1, Uw1, Ub1, MixW1, Mixb1),
              (Mw2, Mb2, Uw2, Ub2, MixW2, Mixb2)]
    degb = None
    for l, (Mw, Mb, Uw, Ub, MixW, Mixb) in enumerate(layers):
        A, B = _tc_proj(h, Mw, Mb)
        S, Q, MX, MN, deg = _sc_segment_stats(A, packed, cnts)
        if degb is None:
            degb = deg[:, None] * jnp.ones((1, D), jnp.float32)
        h = _tc_layer(h, S, Q, MX, MN, degb, B,
                      Uw, Ub, MixW, Mixb, last=(l == 2))

    FCw_pad = jnp.pad(FCw, ((0, 0), (0, D - FCw.shape[1])))
    FCb_pad = jnp.pad(FCb, (0, D - FCb.shape[0]))
    out = _tc_fc(h, FCw_pad, FCb_pad.reshape(1, D))
    return out[:n, :FCw.shape[1]]
